# Initial kernel scaffold; baseline (speedup 1.0000x reference)
import jax
import jax.numpy as jnp
from jax.experimental import pallas as pl


def kernel(x_bat, centroids, centroid_lut, fc3_w, fc3_b, conv_lut, add_lut, fc_lut, relu_lut, w0, w1, w2, w3, b_c1, b_c2, b_f1, b_f2):
    raise NotImplementedError("write your pallas kernel here")


# speedup vs baseline: 1.0589x; 1.0589x over previous
import functools

import jax
import jax.numpy as jnp
from jax.experimental import pallas as pl

_NSYM = 256
_BLK = 512


def _fold_body(n, parts_ref, lut_t_ref, out_ref):
    iota = jax.lax.broadcasted_iota(jnp.int32, (_NSYM, _BLK), 0)
    tmp = parts_ref[0:1, :].astype(jnp.float32)

    def step(j, tmp):
        p = parts_ref[pl.ds(j, 1), :]
        onehot_p = (iota == p).astype(jnp.bfloat16)
        g = jnp.dot(lut_t_ref[...], onehot_p, preferred_element_type=jnp.float32)
        sel = (iota.astype(jnp.float32) == tmp).astype(jnp.float32)
        return jnp.sum(g * sel, axis=0, keepdims=True)

    tmp = jax.lax.fori_loop(1, n, step, tmp)
    out_ref[0, :, :] = tmp


def _lut_fold_pallas(parts, add_lut_t_bf16):
    r, n = parts.shape
    n8 = (n + 7) // 8 * 8
    rp = (r + _BLK - 1) // _BLK * _BLK
    nb = rp // _BLK
    pt = jnp.zeros((n8, rp), jnp.int32)
    pt = pt.at[:n, :r].set(parts.T)
    out = pl.pallas_call(
        functools.partial(_fold_body, n),
        grid=(nb,),
        in_specs=[
            pl.BlockSpec((n8, _BLK), lambda i: (0, i)),
            pl.BlockSpec((_NSYM, _NSYM), lambda i: (0, 0)),
        ],
        out_specs=pl.BlockSpec((1, 1, _BLK), lambda i: (i, 0, 0)),
        out_shape=jax.ShapeDtypeStruct((nb, 1, _BLK), jnp.float32),
    )(pt, add_lut_t_bf16)
    return out.reshape(rp)[:r].astype(jnp.int32)


def _windows(arr, k, s):
    g = arr.shape[0]
    c = arr.shape[2]
    o = (g - k) // s + 1
    rows = (jnp.arange(o) * s)[:, None] + jnp.arange(k)[None, :]
    w = arr[rows][:, :, rows]
    w = jnp.transpose(w, (0, 2, 1, 3, 4))
    return w.reshape(o * o, k * k * c), o


def _discretize(img, centroids):
    win, o = _windows(img[:, :, None], 4, 1)
    d = ((win[:, None, :] - centroids[None, :, :]) ** 2).sum(-1)
    return jnp.argmin(d, axis=-1).reshape(o, o)


def kernel(x_bat, centroids, centroid_lut, fc3_w, fc3_b, conv_lut, add_lut,
           fc_lut, relu_lut, w0, w1, w2, w3, b_c1, b_c2, b_f1, b_f2):
    b = x_bat.shape[0]
    add_t = add_lut.T.astype(jnp.bfloat16)

    def conv_parts(sym_b, ker):
        o = (sym_b.shape[1] - 5) // 2 + 1
        win = jax.vmap(lambda s: _windows(s, 5, 2)[0])(sym_b)
        out_ch = ker.shape[1]
        parts = conv_lut[win[:, :, None, :], ker.T[None, None, :, :]]
        parts = jnp.sort(parts, axis=-1)
        return parts.reshape(b * o * o * out_ch, -1), o, out_ch

    sym = jax.vmap(lambda im: _discretize(im[0], centroids))(x_bat)

    parts, o1, c1 = conv_parts(sym[:, :, :, None], w0)
    tmp = _lut_fold_pallas(parts, add_t).reshape(b, o1 * o1, c1)
    h = b_c1[tmp, jnp.arange(c1)[None, None, :]]
    h = relu_lut[h].reshape(b, o1, o1, c1)

    parts, o2, c2 = conv_parts(h, w1)
    tmp = _lut_fold_pallas(parts, add_t).reshape(b, o2 * o2, c2)
    h = b_c2[tmp, jnp.arange(c2)[None, None, :]]
    h = relu_lut[h].reshape(b, o2, o2, c2)

    flat = jnp.transpose(h, (0, 3, 1, 2)).reshape(b, -1)
    parts = fc_lut[flat[:, None, :], w2[None, :, :]]
    parts = jnp.sort(parts, axis=-1).reshape(b * w2.shape[0], -1)
    f = _lut_fold_pallas(parts, add_t).reshape(b, w2.shape[0])
    f = relu_lut[b_f1[f, jnp.arange(w2.shape[0])[None, :]]]

    parts = fc_lut[f[:, None, :], w3[None, :, :]]
    parts = jnp.sort(parts, axis=-1).reshape(b * w3.shape[0], -1)
    f = _lut_fold_pallas(parts, add_t).reshape(b, w3.shape[0])
    f = relu_lut[b_f2[f, jnp.arange(w3.shape[0])[None, :]]]

    feats = centroid_lut[f]
    logits = feats @ fc3_w.T + fc3_b
    return jax.nn.softmax(logits, axis=1)


# speedup vs baseline: 1.1242x; 1.0617x over previous
import functools

import jax
import jax.numpy as jnp
from jax.experimental import pallas as pl

_NSYM = 256
_BLK = 512


def _fold_body(n, parts_ref, lut_t_ref, out_ref):
    iota = jax.lax.broadcasted_iota(jnp.int32, (_NSYM, _BLK), 0)
    iota_f = iota.astype(jnp.float32)

    def hist(j, cum):
        x = parts_ref[pl.ds(j, 1), :]
        return cum + (iota >= x).astype(jnp.float32)

    cum = jax.lax.fori_loop(0, n, hist, jnp.zeros((_NSYM, _BLK), jnp.float32))

    def rank(i):
        return jnp.sum((cum <= i).astype(jnp.float32), axis=0, keepdims=True)

    tmp = rank(0.0)

    def step(j, tmp):
        p = rank(j.astype(jnp.float32))
        onehot_p = (iota_f == p).astype(jnp.bfloat16)
        g = jnp.dot(lut_t_ref[...], onehot_p, preferred_element_type=jnp.float32)
        sel = (iota_f == tmp).astype(jnp.float32)
        return jnp.sum(g * sel, axis=0, keepdims=True)

    tmp = jax.lax.fori_loop(1, n, step, tmp)
    out_ref[0, :, :] = tmp


def _lut_fold_pallas(parts, add_lut_t_bf16):
    r, n = parts.shape
    n8 = (n + 7) // 8 * 8
    rp = (r + _BLK - 1) // _BLK * _BLK
    nb = rp // _BLK
    pt = jnp.zeros((n8, rp), jnp.int32)
    pt = pt.at[:n, :r].set(parts.T)
    out = pl.pallas_call(
        functools.partial(_fold_body, n),
        grid=(nb,),
        in_specs=[
            pl.BlockSpec((n8, _BLK), lambda i: (0, i)),
            pl.BlockSpec((_NSYM, _NSYM), lambda i: (0, 0)),
        ],
        out_specs=pl.BlockSpec((1, 1, _BLK), lambda i: (i, 0, 0)),
        out_shape=jax.ShapeDtypeStruct((nb, 1, _BLK), jnp.float32),
    )(pt, add_lut_t_bf16)
    return out.reshape(rp)[:r].astype(jnp.int32)


def _windows(arr, k, s):
    g = arr.shape[0]
    c = arr.shape[2]
    o = (g - k) // s + 1
    rows = (jnp.arange(o) * s)[:, None] + jnp.arange(k)[None, :]
    w = arr[rows][:, :, rows]
    w = jnp.transpose(w, (0, 2, 1, 3, 4))
    return w.reshape(o * o, k * k * c), o


def _discretize(img, centroids):
    win, o = _windows(img[:, :, None], 4, 1)
    d = ((win[:, None, :] - centroids[None, :, :]) ** 2).sum(-1)
    return jnp.argmin(d, axis=-1).reshape(o, o)


def kernel(x_bat, centroids, centroid_lut, fc3_w, fc3_b, conv_lut, add_lut,
           fc_lut, relu_lut, w0, w1, w2, w3, b_c1, b_c2, b_f1, b_f2):
    b = x_bat.shape[0]
    add_t = add_lut.T.astype(jnp.bfloat16)

    def conv_parts(sym_b, ker):
        o = (sym_b.shape[1] - 5) // 2 + 1
        win = jax.vmap(lambda s: _windows(s, 5, 2)[0])(sym_b)
        out_ch = ker.shape[1]
        parts = conv_lut[win[:, :, None, :], ker.T[None, None, :, :]]
        return parts.reshape(b * o * o * out_ch, -1), o, out_ch

    sym = jax.vmap(lambda im: _discretize(im[0], centroids))(x_bat)

    parts, o1, c1 = conv_parts(sym[:, :, :, None], w0)
    tmp = _lut_fold_pallas(parts, add_t).reshape(b, o1 * o1, c1)
    h = b_c1[tmp, jnp.arange(c1)[None, None, :]]
    h = relu_lut[h].reshape(b, o1, o1, c1)

    parts, o2, c2 = conv_parts(h, w1)
    tmp = _lut_fold_pallas(parts, add_t).reshape(b, o2 * o2, c2)
    h = b_c2[tmp, jnp.arange(c2)[None, None, :]]
    h = relu_lut[h].reshape(b, o2, o2, c2)

    flat = jnp.transpose(h, (0, 3, 1, 2)).reshape(b, -1)
    parts = fc_lut[flat[:, None, :], w2[None, :, :]].reshape(b * w2.shape[0], -1)
    f = _lut_fold_pallas(parts, add_t).reshape(b, w2.shape[0])
    f = relu_lut[b_f1[f, jnp.arange(w2.shape[0])[None, :]]]

    parts = fc_lut[f[:, None, :], w3[None, :, :]].reshape(b * w3.shape[0], -1)
    f = _lut_fold_pallas(parts, add_t).reshape(b, w3.shape[0])
    f = relu_lut[b_f2[f, jnp.arange(w3.shape[0])[None, :]]]

    feats = centroid_lut[f]
    logits = feats @ fc3_w.T + fc3_b
    return jax.nn.softmax(logits, axis=1)


# speedup vs baseline: 8.1601x; 7.2583x over previous
import functools

import jax
import jax.numpy as jnp
from jax.experimental import pallas as pl

_NSYM = 256
_BLK = 512


def _fold_body(n, ia_ref, ib_ref, tab_t_ref, lut_t_ref, out_ref):
    iota = jax.lax.broadcasted_iota(jnp.int32, (_NSYM, _BLK), 0)
    iota_f = iota.astype(jnp.float32)

    def gather2(t_ref, a, b_onehot_f):
        g = jnp.dot(t_ref[...], (iota == a).astype(jnp.bfloat16),
                    preferred_element_type=jnp.float32)
        return jnp.sum(g * b_onehot_f, axis=0, keepdims=True)

    def hist(j, cum):
        a = ia_ref[pl.ds(j, 1), :]
        b = ib_ref[pl.ds(j, 1), :]
        x = gather2(tab_t_ref, a, (iota == b).astype(jnp.float32))
        return cum + (iota_f >= x).astype(jnp.float32)

    cum = jax.lax.fori_loop(0, n, hist, jnp.zeros((_NSYM, _BLK), jnp.float32))

    def rank(i):
        return jnp.sum((cum <= i).astype(jnp.float32), axis=0, keepdims=True)

    tmp = rank(0.0)

    def step(j, tmp):
        p = rank(j.astype(jnp.float32))
        tmp = gather2(lut_t_ref, p.astype(jnp.int32),
                      (iota_f == tmp).astype(jnp.float32))
        return tmp

    tmp = jax.lax.fori_loop(1, n, step, tmp)
    out_ref[0, :, :] = tmp


def _lut_fold_pallas(idx_a, idx_b, tab_t, add_t):
    r, n = idx_a.shape
    n8 = (n + 7) // 8 * 8
    rp = (r + _BLK - 1) // _BLK * _BLK
    nb = rp // _BLK
    pad = lambda x: jnp.zeros((n8, rp), jnp.int32).at[:n, :r].set(x.T)
    out = pl.pallas_call(
        functools.partial(_fold_body, n),
        grid=(nb,),
        in_specs=[
            pl.BlockSpec((n8, _BLK), lambda i: (0, i)),
            pl.BlockSpec((n8, _BLK), lambda i: (0, i)),
            pl.BlockSpec((_NSYM, _NSYM), lambda i: (0, 0)),
            pl.BlockSpec((_NSYM, _NSYM), lambda i: (0, 0)),
        ],
        out_specs=pl.BlockSpec((1, 1, _BLK), lambda i: (i, 0, 0)),
        out_shape=jax.ShapeDtypeStruct((nb, 1, _BLK), jnp.float32),
    )(pad(idx_a), pad(idx_b), tab_t, add_t)
    return out.reshape(rp)[:r].astype(jnp.int32)


def _windows(arr, k, s):
    g = arr.shape[0]
    c = arr.shape[2]
    o = (g - k) // s + 1
    rows = (jnp.arange(o) * s)[:, None] + jnp.arange(k)[None, :]
    w = arr[rows][:, :, rows]
    w = jnp.transpose(w, (0, 2, 1, 3, 4))
    return w.reshape(o * o, k * k * c), o


def _discretize(img, centroids):
    win, o = _windows(img[:, :, None], 4, 1)
    d = ((win[:, None, :] - centroids[None, :, :]) ** 2).sum(-1)
    return jnp.argmin(d, axis=-1).reshape(o, o)


def kernel(x_bat, centroids, centroid_lut, fc3_w, fc3_b, conv_lut, add_lut,
           fc_lut, relu_lut, w0, w1, w2, w3, b_c1, b_c2, b_f1, b_f2):
    b = x_bat.shape[0]
    add_t = add_lut.T.astype(jnp.bfloat16)
    conv_t = conv_lut.T.astype(jnp.bfloat16)
    fc_t = fc_lut.T.astype(jnp.bfloat16)

    def conv_stage(sym_b, ker, bias_lut):
        o = (sym_b.shape[1] - 5) // 2 + 1
        win = jax.vmap(lambda s: _windows(s, 5, 2)[0])(sym_b)
        out_ch = ker.shape[1]
        n = win.shape[-1]
        ia = jnp.broadcast_to(win[:, :, None, :], (b, o * o, out_ch, n))
        ib = jnp.broadcast_to(ker.T[None, None, :, :], (b, o * o, out_ch, n))
        tmp = _lut_fold_pallas(ia.reshape(-1, n), ib.reshape(-1, n),
                               conv_t, add_t).reshape(b, o * o, out_ch)
        h = bias_lut[tmp, jnp.arange(out_ch)[None, None, :]]
        return relu_lut[h].reshape(b, o, o, out_ch)

    def fc_stage(x, w, bias_lut):
        out, n = w.shape
        ia = jnp.broadcast_to(x[:, None, :], (b, out, n))
        ib = jnp.broadcast_to(w[None, :, :], (b, out, n))
        f = _lut_fold_pallas(ia.reshape(-1, n), ib.reshape(-1, n),
                             fc_t, add_t).reshape(b, out)
        return relu_lut[bias_lut[f, jnp.arange(out)[None, :]]]

    sym = jax.vmap(lambda im: _discretize(im[0], centroids))(x_bat)

    h = conv_stage(sym[:, :, :, None], w0, b_c1)
    h = conv_stage(h, w1, b_c2)
    flat = jnp.transpose(h, (0, 3, 1, 2)).reshape(b, -1)
    f = fc_stage(flat, w2, b_f1)
    f = fc_stage(f, w3, b_f2)

    feats = centroid_lut[f]
    logits = feats @ fc3_w.T + fc3_b
    return jax.nn.softmax(logits, axis=1)
